# baseline (speedup 1.0000x reference)
import functools

import jax
import jax.numpy as jnp
from jax import lax
from jax.experimental import pallas as pl
from jax.experimental.pallas import tpu as pltpu
from jax.experimental.pallas import tpu_sc as plsc

N = 10000
E = 320000
F_IN = 128
H = 128
C = 16
G = 64

NC = 2
NS = 16
NW = NC * NS
CH = 128
R = 4
K = -(-E // (NW * CH * R))
EPT = K * R * CH
E_PAD = NW * EPT
ACC_ROWS = 10240
ZCH = ACC_ROWS // NS // CH
DW = 16

_mesh = plsc.VectorSubcoreMesh(
    core_axis_name="c", subcore_axis_name="s", num_cores=NC, num_subcores=NS)

_f32 = jnp.float32


def _hist_body(dstg_hbm, out_hbm, dst_v, hist_v):
  cid = lax.axis_index("c")
  sid = lax.axis_index("s")
  wid = sid * NC + cid
  pltpu.sync_copy(dstg_hbm.at[wid], dst_v)
  zeros16 = jnp.zeros((16,), _f32)
  ones16 = jnp.ones((16,), _f32)

  def zh(i, _):
    hist_v[pl.ds(i * 16, 16)] = zeros16
    return 0
  lax.fori_loop(0, ACC_ROWS // 16, zh, 0)

  def step(j, _):
    def inner(c, _):
      idx = dst_v[j, pl.ds(c * 16, 16)]
      plsc.addupdate_scatter(hist_v, [idx], ones16)
      return 0
    return lax.fori_loop(0, CH // 16, inner, 0)
  lax.fori_loop(0, K * R, step, 0)
  pltpu.sync_copy(hist_v, out_hbm.at[pl.ds(wid * ACC_ROWS, ACC_ROWS)])


_hist_call = pl.kernel(
    _hist_body,
    out_type=jax.ShapeDtypeStruct((NW * ACC_ROWS,), _f32),
    mesh=_mesh,
    scratch_types=[
        pltpu.VMEM((K * R, CH), jnp.int32),
        pltpu.VMEM((ACC_ROWS,), _f32),
    ],
    compiler_params=pltpu.CompilerParams(needs_layout_passes=False),
)


NBUF = 2
IH = (K * R) // 2


def _agg_body(xs_hbm, srcg_hbm, dstg_hbm, zeros_hbm, out_hbm,
              src_v, dst_v, rowsb, acc_sh, g0, g1, s0, s1):
  cid = lax.axis_index("c")
  sid = lax.axis_index("s")
  wid = sid * NC + cid
  gsems = (g0, g1)
  ssems = (s0, s1)
  pltpu.sync_copy(zeros_hbm, rowsb.at[0])

  def zacc(t, _):
    pltpu.sync_copy(rowsb.at[0], acc_sh.at[pl.ds((sid * ZCH + t) * CH, CH)])
    return 0
  lax.fori_loop(0, ZCH, zacc, 0)
  plsc.subcore_barrier()

  for half in range(2):
    pltpu.sync_copy(srcg_hbm.at[wid, pl.ds(half * IH, IH)], src_v)
    pltpu.sync_copy(dstg_hbm.at[wid, pl.ds(half * IH, IH)], dst_v)

    def round_(r, _):
      j0 = r * NBUF
      gds = [pltpu.async_copy(xs_hbm.at[src_v.at[j0 + b]], rowsb.at[b],
                              gsems[b]) for b in range(NBUF)]
      sds = []
      for b in range(NBUF):
        gds[b].wait()
        sds.append(pltpu.async_copy(rowsb.at[b], acc_sh.at[dst_v.at[j0 + b]],
                                    ssems[b], add=True))
      for d in sds:
        d.wait()
      return 0
    lax.fori_loop(0, IH // NBUF, round_, 0)
  plsc.subcore_barrier()

  def cout(t, _):
    r0 = (sid * ZCH + t) * CH
    pltpu.sync_copy(acc_sh.at[pl.ds(r0, CH)], rowsb.at[0])
    pltpu.sync_copy(rowsb.at[0], out_hbm.at[cid, pl.ds(r0, CH)])
    return 0
  lax.fori_loop(0, ZCH, cout, 0)


_agg_call = pl.kernel(
    _agg_body,
    out_type=jax.ShapeDtypeStruct((NC, ACC_ROWS, H), _f32),
    mesh=_mesh,
    scratch_types=[
        pltpu.VMEM((IH, CH), jnp.int32),
        pltpu.VMEM((IH, CH), jnp.int32),
        pltpu.VMEM((NBUF, CH, H), _f32),
        pltpu.VMEM_SHARED((ACC_ROWS, H), _f32),
        pltpu.SemaphoreType.DMA,
        pltpu.SemaphoreType.DMA,
        pltpu.SemaphoreType.DMA,
        pltpu.SemaphoreType.DMA,
    ],
)

_DOT = dict(preferred_element_type=_f32, precision=lax.Precision.HIGHEST)


def _tcd_body(degp_ref, dinv_ref):
  dinv_ref[...] = lax.rsqrt(jnp.sum(degp_ref[...], axis=0) + 1.0)


def _tcd_call(degp):
  return pl.pallas_call(
      _tcd_body,
      out_shape=jax.ShapeDtypeStruct((ACC_ROWS // CH, CH), _f32),
  )(degp)


def _tc0_body(dinv_ref, x_ref, w0_ref, xs0_ref):
  xw = jnp.dot(x_ref[...], w0_ref[...], **_DOT)
  xs0_ref[...] = xw * dinv_ref[...]


def _tc0_call(dinv, x, w0):
  return pl.pallas_call(
      _tc0_body,
      out_shape=jax.ShapeDtypeStruct((N, H), _f32),
  )(dinv, x, w0)


def _tc_mid_body(relu, sp_ref, xs_ref, dinv_ref, b_ref, w_ref, out_ref):
  dinv = dinv_ref[...]
  h = dinv * (sp_ref[0, :N] + sp_ref[1, :N] + xs_ref[...]) + b_ref[...]
  if relu:
    h = jnp.maximum(h, 0.0)
  out_ref[...] = dinv * jnp.dot(h, w_ref[...], **_DOT)


def _tc_mid_call(relu, sp, xs, dinv, b, w):
  return pl.pallas_call(
      functools.partial(_tc_mid_body, relu),
      out_shape=jax.ShapeDtypeStruct((N, H), _f32),
  )(sp, xs, dinv, b, w)


def _tc3_body(sp_ref, xs_ref, dinv_ref, b_ref, batch_ref, wlin_ref, blin_ref,
              y_ref, gm_ref):
  dinv = dinv_ref[...]
  h = jnp.maximum(dinv * (sp_ref[0, :N] + sp_ref[1, :N] + xs_ref[...]) + b_ref[...],
                  0.0)
  gids = batch_ref[...]
  seg_ids = lax.broadcasted_iota(jnp.int32, (1, G), 1)
  onehot = (gids == seg_ids).astype(_f32)
  seg = lax.dot_general(onehot, h, (((0,), (0,)), ((), ())), **_DOT)
  cnt = lax.dot_general(onehot, jnp.ones((N, 1), _f32),
                        (((0,), (0,)), ((), ())), **_DOT)
  gm = seg / jnp.maximum(cnt, 1.0)
  gm_ref[...] = gm
  y_ref[...] = jnp.dot(gm, wlin_ref[...], **_DOT) + blin_ref[...]


def _tc3_call(sp, xs, dinv, b, batch2d, wlin, blin):
  return pl.pallas_call(
      _tc3_body,
      out_shape=[jax.ShapeDtypeStruct((G, C), _f32),
                 jax.ShapeDtypeStruct((G, H), _f32)],
  )(sp, xs, dinv, b, batch2d, wlin, blin)


def kernel(x, edge_index, batch, W0, b0, W1, b1, W2, b2, Wlin, blin):
  src = edge_index[0]
  dst = edge_index[1]
  pad = E_PAD - E
  srcg = jnp.concatenate([src, jnp.zeros((pad,), jnp.int32)]).reshape(NW, K * R, CH)
  dstg = jnp.concatenate([dst, jnp.full((pad,), N, jnp.int32)]).reshape(NW, K * R, CH)
  zrows = jnp.zeros((CH, H), _f32)

  degp_rows = _agg_call(jnp.ones((N, H), _f32), srcg, dstg, zrows)
  degp = degp_rows[:, :, 0].reshape(NC, ACC_ROWS // CH, CH)
  dinv = _tcd_call(degp).reshape(ACC_ROWS, 1)[:N]
  xs = _tc0_call(dinv, x, W0)
  for b, w, relu in ((b0, W1, False), (b1, W2, True)):
    sp = _agg_call(xs, srcg, dstg, zrows)
    xs = _tc_mid_call(relu, sp, xs, dinv, b.reshape(1, H), w)
  sp = _agg_call(xs, srcg, dstg, zrows)
  y, gm = _tc3_call(sp, xs, dinv, b2.reshape(1, H), batch.reshape(N, 1),
                    Wlin, blin.reshape(1, C))
  return (y, gm)


# speedup vs baseline: 2.0964x; 2.0964x over previous
import functools

import jax
import jax.numpy as jnp
from jax import lax
from jax.experimental import pallas as pl
from jax.experimental.pallas import tpu as pltpu
from jax.experimental.pallas import tpu_sc as plsc

N = 10000
E = 320000
F_IN = 128
H = 128
C = 16
G = 64

NC = 2
NS = 16
NW = NC * NS
CH = 128
CHUNKS = -(-E // (NW * CH))
EPT = CHUNKS * CH
E_PAD = NW * EPT
ACC_ROWS = 10240
ZCH = ACC_ROWS // NS // CH
DW = 16

_mesh = plsc.VectorSubcoreMesh(
    core_axis_name="c", subcore_axis_name="s", num_cores=NC, num_subcores=NS)

_f32 = jnp.float32


def _hist_body(dstg_hbm, out_hbm, dst_v, hist_v):
  cid = lax.axis_index("c")
  sid = lax.axis_index("s")
  wid = sid * NC + cid
  pltpu.sync_copy(dstg_hbm.at[wid], dst_v)
  zeros16 = jnp.zeros((16,), _f32)
  ones16 = jnp.ones((16,), _f32)

  def zh(i, _):
    hist_v[pl.ds(i * 16, 16)] = zeros16
    return 0
  lax.fori_loop(0, ACC_ROWS // 16, zh, 0)

  def step(j, _):
    def inner(c, _):
      idx = dst_v[j, pl.ds(c * 16, 16)]
      plsc.addupdate_scatter(hist_v, [idx], ones16)
      return 0
    return lax.fori_loop(0, CH // 16, inner, 0)
  lax.fori_loop(0, CHUNKS, step, 0)
  pltpu.sync_copy(hist_v, out_hbm.at[pl.ds(wid * ACC_ROWS, ACC_ROWS)])


_hist_call = pl.kernel(
    _hist_body,
    out_type=jax.ShapeDtypeStruct((NW * ACC_ROWS,), _f32),
    mesh=_mesh,
    scratch_types=[
        pltpu.VMEM((CHUNKS, CH), jnp.int32),
        pltpu.VMEM((ACC_ROWS,), _f32),
    ],
    compiler_params=pltpu.CompilerParams(needs_layout_passes=False),
)


def _agg_body(xs_hbm, srcg_hbm, dstg_hbm, zeros_hbm, out_hbm,
              src_v, dst_v, rows_v, acc_sh, sem):
  cid = lax.axis_index("c")
  sid = lax.axis_index("s")
  wid = sid * NC + cid
  pltpu.sync_copy(zeros_hbm, rows_v)
  pltpu.sync_copy(srcg_hbm.at[wid], src_v)
  pltpu.sync_copy(dstg_hbm.at[wid], dst_v)

  def zacc(t, _):
    pltpu.sync_copy(rows_v, acc_sh.at[pl.ds((sid * ZCH + t) * CH, CH)])
    return 0
  lax.fori_loop(0, ZCH, zacc, 0)
  plsc.subcore_barrier()

  def step(j, _):
    pltpu.async_copy(xs_hbm.at[src_v.at[j]], rows_v, sem).wait()
    pltpu.sync_copy(rows_v, acc_sh.at[dst_v.at[j]], add=True)
    return 0
  lax.fori_loop(0, CHUNKS, step, 0)
  plsc.subcore_barrier()

  def cout(t, _):
    r0 = (sid * ZCH + t) * CH
    pltpu.sync_copy(acc_sh.at[pl.ds(r0, CH)], rows_v)
    pltpu.sync_copy(rows_v, out_hbm.at[cid, pl.ds(r0, CH)])
    return 0
  lax.fori_loop(0, ZCH, cout, 0)


_agg_call = pl.kernel(
    _agg_body,
    out_type=jax.ShapeDtypeStruct((NC, ACC_ROWS, H), _f32),
    mesh=_mesh,
    scratch_types=[
        pltpu.VMEM((CHUNKS, CH), jnp.int32),
        pltpu.VMEM((CHUNKS, CH), jnp.int32),
        pltpu.VMEM((CH, H), _f32),
        pltpu.VMEM_SHARED((ACC_ROWS, H), _f32),
        pltpu.SemaphoreType.DMA,
    ],
)

_DOT = dict(preferred_element_type=_f32, precision=lax.Precision.HIGHEST)


def _tcd_body(degp_ref, dinv_ref):
  dinv_ref[...] = lax.rsqrt(jnp.sum(degp_ref[...], axis=0) + 1.0)


def _tcd_call(degp):
  return pl.pallas_call(
      _tcd_body,
      out_shape=jax.ShapeDtypeStruct((ACC_ROWS // CH, CH), _f32),
  )(degp)


def _tc0_body(dinv_ref, x_ref, w0_ref, xs0_ref):
  xw = jnp.dot(x_ref[...], w0_ref[...], **_DOT)
  xs0_ref[...] = xw * dinv_ref[...]


def _tc0_call(dinv, x, w0):
  return pl.pallas_call(
      _tc0_body,
      out_shape=jax.ShapeDtypeStruct((N, H), _f32),
  )(dinv, x, w0)


def _tc_mid_body(relu, sp_ref, xs_ref, dinv_ref, b_ref, w_ref, out_ref):
  dinv = dinv_ref[...]
  h = dinv * (sp_ref[0, :N] + sp_ref[1, :N] + xs_ref[...]) + b_ref[...]
  if relu:
    h = jnp.maximum(h, 0.0)
  out_ref[...] = dinv * jnp.dot(h, w_ref[...], **_DOT)


def _tc_mid_call(relu, sp, xs, dinv, b, w):
  return pl.pallas_call(
      functools.partial(_tc_mid_body, relu),
      out_shape=jax.ShapeDtypeStruct((N, H), _f32),
  )(sp, xs, dinv, b, w)


def _tc3_body(sp_ref, xs_ref, dinv_ref, b_ref, batch_ref, wlin_ref, blin_ref,
              y_ref, gm_ref):
  dinv = dinv_ref[...]
  h = jnp.maximum(dinv * (sp_ref[0, :N] + sp_ref[1, :N] + xs_ref[...]) + b_ref[...],
                  0.0)
  gids = batch_ref[...]
  seg_ids = lax.broadcasted_iota(jnp.int32, (1, G), 1)
  onehot = (gids == seg_ids).astype(_f32)
  seg = lax.dot_general(onehot, h, (((0,), (0,)), ((), ())), **_DOT)
  cnt = lax.dot_general(onehot, jnp.ones((N, 1), _f32),
                        (((0,), (0,)), ((), ())), **_DOT)
  gm = seg / jnp.maximum(cnt, 1.0)
  gm_ref[...] = gm
  y_ref[...] = jnp.dot(gm, wlin_ref[...], **_DOT) + blin_ref[...]


def _tc3_call(sp, xs, dinv, b, batch2d, wlin, blin):
  return pl.pallas_call(
      _tc3_body,
      out_shape=[jax.ShapeDtypeStruct((G, C), _f32),
                 jax.ShapeDtypeStruct((G, H), _f32)],
  )(sp, xs, dinv, b, batch2d, wlin, blin)


def kernel(x, edge_index, batch, W0, b0, W1, b1, W2, b2, Wlin, blin):
  src = edge_index[0]
  dst = edge_index[1]
  pad = E_PAD - E
  pad_dst = N + (jnp.arange(pad, dtype=jnp.int32) % (ACC_ROWS - N))
  srcg = jnp.concatenate([src, jnp.zeros((pad,), jnp.int32)]).reshape(NW, CHUNKS, CH)
  dstg = jnp.concatenate([dst, pad_dst]).reshape(NW, CHUNKS, CH)
  zrows = jnp.zeros((CH, H), _f32)

  degp = _hist_call(dstg).reshape(NW, ACC_ROWS // CH, CH)
  dinv = _tcd_call(degp).reshape(ACC_ROWS, 1)[:N]
  xs = _tc0_call(dinv, x, W0)
  for b, w, relu in ((b0, W1, False), (b1, W2, True)):
    sp = _agg_call(xs, srcg, dstg, zrows)
    xs = _tc_mid_call(relu, sp, xs, dinv, b.reshape(1, H), w)
  sp = _agg_call(xs, srcg, dstg, zrows)
  y, gm = _tc3_call(sp, xs, dinv, b2.reshape(1, H), batch.reshape(N, 1),
                    Wlin, blin.reshape(1, C))
  return (y, gm)


# speedup vs baseline: 2.2267x; 1.0622x over previous
import functools

import jax
import jax.numpy as jnp
from jax import lax
from jax.experimental import pallas as pl
from jax.experimental.pallas import tpu as pltpu
from jax.experimental.pallas import tpu_sc as plsc

N = 10000
E = 320000
F_IN = 128
H = 128
C = 16
G = 64

NC = 2
NS = 16
NW = NC * NS
CH = 128
CHUNKS = -(-E // (NW * CH))
EPT = CHUNKS * CH
E_PAD = NW * EPT
ACC_ROWS = 10240
ZCH = ACC_ROWS // NS // CH
DW = 16

_mesh = plsc.VectorSubcoreMesh(
    core_axis_name="c", subcore_axis_name="s", num_cores=NC, num_subcores=NS)

_f32 = jnp.float32


def _hist_body(dstg_hbm, out_hbm, dst_v, hist_v):
  cid = lax.axis_index("c")
  sid = lax.axis_index("s")
  wid = sid * NC + cid
  pltpu.sync_copy(dstg_hbm.at[wid], dst_v)
  zeros16 = jnp.zeros((16,), _f32)
  ones16 = jnp.ones((16,), _f32)

  def zh(i, _):
    hist_v[pl.ds(i * 16, 16)] = zeros16
    return 0
  lax.fori_loop(0, ACC_ROWS // 16, zh, 0)

  def step(j, _):
    def inner(c, _):
      idx = dst_v[j, pl.ds(c * 16, 16)]
      plsc.addupdate_scatter(hist_v, [idx], ones16)
      return 0
    return lax.fori_loop(0, CH // 16, inner, 0)
  lax.fori_loop(0, CHUNKS, step, 0)
  pltpu.sync_copy(hist_v, out_hbm.at[pl.ds(wid * ACC_ROWS, ACC_ROWS)])


_hist_call = pl.kernel(
    _hist_body,
    out_type=jax.ShapeDtypeStruct((NW * ACC_ROWS,), _f32),
    mesh=_mesh,
    scratch_types=[
        pltpu.VMEM((CHUNKS, CH), jnp.int32),
        pltpu.VMEM((ACC_ROWS,), _f32),
    ],
    compiler_params=pltpu.CompilerParams(needs_layout_passes=False),
)


NBUF = 2
IH = 40


def _agg_body(xs_hbm, srcg_hbm, dstg_hbm, zeros_hbm, out_hbm,
              src_v, dst_v, rowsb, acc_sh, g0, g1, s0, s1):
  cid = lax.axis_index("c")
  sid = lax.axis_index("s")
  wid = sid * NC + cid
  gsems = (g0, g1)
  ssems = (s0, s1)
  pltpu.sync_copy(zeros_hbm, rowsb.at[0])

  def zacc(t, _):
    pltpu.sync_copy(rowsb.at[0], acc_sh.at[pl.ds((sid * ZCH + t) * CH, CH)])
    return 0
  lax.fori_loop(0, ZCH, zacc, 0)
  plsc.subcore_barrier()

  for half in range(2):
    hn = IH if half == 0 else CHUNKS - IH
    pltpu.sync_copy(srcg_hbm.at[wid, pl.ds(half * IH, hn)],
                    src_v.at[pl.ds(0, hn)])
    pltpu.sync_copy(dstg_hbm.at[wid, pl.ds(half * IH, hn)],
                    dst_v.at[pl.ds(0, hn)])

    def round_(r, _):
      j0 = r * NBUF
      gds = [pltpu.async_copy(xs_hbm.at[src_v.at[j0 + b]], rowsb.at[b],
                              gsems[b]) for b in range(NBUF)]
      sds = []
      for b in range(NBUF):
        gds[b].wait()
        sds.append(pltpu.async_copy(rowsb.at[b], acc_sh.at[dst_v.at[j0 + b]],
                                    ssems[b], add=True))
      for d in sds:
        d.wait()
      return 0
    lax.fori_loop(0, hn // NBUF, round_, 0)
    if hn % NBUF:
      j = hn - 1
      pltpu.async_copy(xs_hbm.at[src_v.at[j]], rowsb.at[0], gsems[0]).wait()
      pltpu.sync_copy(rowsb.at[0], acc_sh.at[dst_v.at[j]], add=True)
  plsc.subcore_barrier()

  def cout(t, _):
    r0 = (sid * ZCH + t) * CH
    pltpu.sync_copy(acc_sh.at[pl.ds(r0, CH)], rowsb.at[0])
    pltpu.sync_copy(rowsb.at[0], out_hbm.at[cid, pl.ds(r0, CH)])
    return 0
  lax.fori_loop(0, ZCH, cout, 0)


_agg_call = pl.kernel(
    _agg_body,
    out_type=jax.ShapeDtypeStruct((NC, ACC_ROWS, H), _f32),
    mesh=_mesh,
    scratch_types=[
        pltpu.VMEM((IH, CH), jnp.int32),
        pltpu.VMEM((IH, CH), jnp.int32),
        pltpu.VMEM((NBUF, CH, H), _f32),
        pltpu.VMEM_SHARED((ACC_ROWS, H), _f32),
        pltpu.SemaphoreType.DMA,
        pltpu.SemaphoreType.DMA,
        pltpu.SemaphoreType.DMA,
        pltpu.SemaphoreType.DMA,
    ],
)

_DOT = dict(preferred_element_type=_f32, precision=lax.Precision.HIGHEST)


def _tcd_body(degp_ref, dinv_ref):
  dinv_ref[...] = lax.rsqrt(jnp.sum(degp_ref[...], axis=0) + 1.0)


def _tcd_call(degp):
  return pl.pallas_call(
      _tcd_body,
      out_shape=jax.ShapeDtypeStruct((ACC_ROWS // CH, CH), _f32),
  )(degp)


def _tc0_body(dinv_ref, x_ref, w0_ref, xs0_ref):
  xw = jnp.dot(x_ref[...], w0_ref[...], **_DOT)
  xs0_ref[...] = xw * dinv_ref[...]


def _tc0_call(dinv, x, w0):
  return pl.pallas_call(
      _tc0_body,
      out_shape=jax.ShapeDtypeStruct((N, H), _f32),
  )(dinv, x, w0)


def _tc_mid_body(relu, sp_ref, xs_ref, dinv_ref, b_ref, w_ref, out_ref):
  dinv = dinv_ref[...]
  h = dinv * (sp_ref[0, :N] + sp_ref[1, :N] + xs_ref[...]) + b_ref[...]
  if relu:
    h = jnp.maximum(h, 0.0)
  out_ref[...] = dinv * jnp.dot(h, w_ref[...], **_DOT)


def _tc_mid_call(relu, sp, xs, dinv, b, w):
  return pl.pallas_call(
      functools.partial(_tc_mid_body, relu),
      out_shape=jax.ShapeDtypeStruct((N, H), _f32),
  )(sp, xs, dinv, b, w)


def _tc3_body(sp_ref, xs_ref, dinv_ref, b_ref, batch_ref, wlin_ref, blin_ref,
              y_ref, gm_ref):
  dinv = dinv_ref[...]
  h = jnp.maximum(dinv * (sp_ref[0, :N] + sp_ref[1, :N] + xs_ref[...]) + b_ref[...],
                  0.0)
  gids = batch_ref[...]
  seg_ids = lax.broadcasted_iota(jnp.int32, (1, G), 1)
  onehot = (gids == seg_ids).astype(_f32)
  seg = lax.dot_general(onehot, h, (((0,), (0,)), ((), ())), **_DOT)
  cnt = lax.dot_general(onehot, jnp.ones((N, 1), _f32),
                        (((0,), (0,)), ((), ())), **_DOT)
  gm = seg / jnp.maximum(cnt, 1.0)
  gm_ref[...] = gm
  y_ref[...] = jnp.dot(gm, wlin_ref[...], **_DOT) + blin_ref[...]


def _tc3_call(sp, xs, dinv, b, batch2d, wlin, blin):
  return pl.pallas_call(
      _tc3_body,
      out_shape=[jax.ShapeDtypeStruct((G, C), _f32),
                 jax.ShapeDtypeStruct((G, H), _f32)],
  )(sp, xs, dinv, b, batch2d, wlin, blin)


def kernel(x, edge_index, batch, W0, b0, W1, b1, W2, b2, Wlin, blin):
  src = edge_index[0]
  dst = edge_index[1]
  pad = E_PAD - E
  pad_dst = N + (jnp.arange(pad, dtype=jnp.int32) % (ACC_ROWS - N))
  srcg = jnp.concatenate([src, jnp.zeros((pad,), jnp.int32)]).reshape(NW, CHUNKS, CH)
  dstg = jnp.concatenate([dst, pad_dst]).reshape(NW, CHUNKS, CH)
  zrows = jnp.zeros((CH, H), _f32)

  degp = _hist_call(dstg).reshape(NW, ACC_ROWS // CH, CH)
  dinv = _tcd_call(degp).reshape(ACC_ROWS, 1)[:N]
  xs = _tc0_call(dinv, x, W0)
  for b, w, relu in ((b0, W1, False), (b1, W2, True)):
    sp = _agg_call(xs, srcg, dstg, zrows)
    xs = _tc_mid_call(relu, sp, xs, dinv, b.reshape(1, H), w)
  sp = _agg_call(xs, srcg, dstg, zrows)
  y, gm = _tc3_call(sp, xs, dinv, b2.reshape(1, H), batch.reshape(N, 1),
                    Wlin, blin.reshape(1, C))
  return (y, gm)


# speedup vs baseline: 4.1628x; 1.8695x over previous
import functools

import jax
import jax.numpy as jnp
from jax import lax
from jax.experimental import pallas as pl
from jax.experimental.pallas import tpu as pltpu
from jax.experimental.pallas import tpu_sc as plsc

N = 10000
E = 320000
F_IN = 128
H = 128
C = 16
G = 64

NC = 2
NS = 16
NW = NC * NS
CH = 128
CHUNKS = -(-E // (NW * CH))
EPT = CHUNKS * CH
E_PAD = NW * EPT
ACC_ROWS = 10240
ZCH = ACC_ROWS // NS // CH
DW = 16

_mesh = plsc.VectorSubcoreMesh(
    core_axis_name="c", subcore_axis_name="s", num_cores=NC, num_subcores=NS)

_f32 = jnp.float32


def _hist_body(dstg_hbm, out_hbm, dst_v, hist_v):
  cid = lax.axis_index("c")
  sid = lax.axis_index("s")
  wid = sid * NC + cid
  pltpu.sync_copy(dstg_hbm.at[wid], dst_v)
  zeros16 = jnp.zeros((16,), _f32)
  ones16 = jnp.ones((16,), _f32)

  def zh(i, _):
    hist_v[pl.ds(i * 16, 16)] = zeros16
    return 0
  lax.fori_loop(0, ACC_ROWS // 16, zh, 0)

  def step(j, _):
    def inner(c, _):
      idx = dst_v[j, pl.ds(c * 16, 16)]
      plsc.addupdate_scatter(hist_v, [idx], ones16)
      return 0
    return lax.fori_loop(0, CH // 16, inner, 0)
  lax.fori_loop(0, CHUNKS, step, 0)
  pltpu.sync_copy(hist_v, out_hbm.at[pl.ds(wid * ACC_ROWS, ACC_ROWS)])


_hist_call = pl.kernel(
    _hist_body,
    out_type=jax.ShapeDtypeStruct((NW * ACC_ROWS,), _f32),
    mesh=_mesh,
    scratch_types=[
        pltpu.VMEM((CHUNKS, CH), jnp.int32),
        pltpu.VMEM((ACC_ROWS,), _f32),
    ],
    compiler_params=pltpu.CompilerParams(needs_layout_passes=False),
)


NBUF = 2
IH = 40


def _agg_body(xs_hbm, srcg_hbm, dstg_hbm, zeros_hbm, out_hbm,
              src_v, dst_v, rowsb, acc_sh, g0, g1, s0, s1):
  cid = lax.axis_index("c")
  sid = lax.axis_index("s")
  wid = sid * NC + cid
  gsems = (g0, g1)
  ssems = (s0, s1)
  pltpu.sync_copy(zeros_hbm, rowsb.at[0])

  def zacc(t, _):
    pltpu.sync_copy(rowsb.at[0], acc_sh.at[pl.ds((sid * ZCH + t) * CH, CH)])
    return 0
  lax.fori_loop(0, ZCH, zacc, 0)
  plsc.subcore_barrier()

  for half in range(2):
    hn = IH if half == 0 else CHUNKS - IH
    pltpu.sync_copy(srcg_hbm.at[wid, pl.ds(half * IH, hn)],
                    src_v.at[pl.ds(0, hn)])
    pltpu.sync_copy(dstg_hbm.at[wid, pl.ds(half * IH, hn)],
                    dst_v.at[pl.ds(0, hn)])

    def round_(r, _):
      j0 = r * NBUF
      gds = [pltpu.async_copy(xs_hbm.at[src_v.at[j0 + b]], rowsb.at[b],
                              gsems[b]) for b in range(NBUF)]
      sds = []
      for b in range(NBUF):
        gds[b].wait()
        sds.append(pltpu.async_copy(rowsb.at[b], acc_sh.at[dst_v.at[j0 + b]],
                                    ssems[b], add=True))
      for d in sds:
        d.wait()
      return 0
    lax.fori_loop(0, hn // NBUF, round_, 0)
    if hn % NBUF:
      j = hn - 1
      pltpu.async_copy(xs_hbm.at[src_v.at[j]], rowsb.at[0], gsems[0]).wait()
      pltpu.sync_copy(rowsb.at[0], acc_sh.at[dst_v.at[j]], add=True)
  plsc.subcore_barrier()

  def cout(t, _):
    r0 = (sid * ZCH + t) * CH
    pltpu.sync_copy(acc_sh.at[pl.ds(r0, CH)], rowsb.at[0])
    pltpu.sync_copy(rowsb.at[0], out_hbm.at[cid, pl.ds(r0, CH)])
    return 0
  lax.fori_loop(0, ZCH, cout, 0)


_agg_call = pl.kernel(
    _agg_body,
    out_type=jax.ShapeDtypeStruct((NC, ACC_ROWS, H), _f32),
    mesh=_mesh,
    scratch_types=[
        pltpu.VMEM((IH, CH), jnp.int32),
        pltpu.VMEM((IH, CH), jnp.int32),
        pltpu.VMEM((NBUF, CH, H), _f32),
        pltpu.VMEM_SHARED((ACC_ROWS, H), _f32),
        pltpu.SemaphoreType.DMA,
        pltpu.SemaphoreType.DMA,
        pltpu.SemaphoreType.DMA,
        pltpu.SemaphoreType.DMA,
    ],
)

_DOT = dict(preferred_element_type=_f32, precision=lax.Precision.HIGHEST)


def _tcd_body(degp_ref, dinv_ref):
  dinv_ref[...] = lax.rsqrt(jnp.sum(degp_ref[...], axis=0) + 1.0)


def _tcd_call(degp):
  return pl.pallas_call(
      _tcd_body,
      out_shape=jax.ShapeDtypeStruct((ACC_ROWS // CH, CH), _f32),
  )(degp)


def _tc0_body(dinv_ref, x_ref, w0_ref, xs0_ref):
  xw = jnp.dot(x_ref[...], w0_ref[...], **_DOT)
  xs0_ref[...] = xw * dinv_ref[...]


def _tc0_call(dinv, x, w0):
  return pl.pallas_call(
      _tc0_body,
      out_shape=jax.ShapeDtypeStruct((N, H), _f32),
  )(dinv, x, w0)


def _tc_mid_body(relu, sp_ref, xs_ref, dinv_ref, b_ref, w_ref, out_ref):
  dinv = dinv_ref[...]
  h = dinv * (sp_ref[0, :N] + sp_ref[1, :N] + xs_ref[...]) + b_ref[...]
  if relu:
    h = jnp.maximum(h, 0.0)
  out_ref[...] = dinv * jnp.dot(h, w_ref[...], **_DOT)


def _tc_mid_call(relu, sp, xs, dinv, b, w):
  return pl.pallas_call(
      functools.partial(_tc_mid_body, relu),
      out_shape=jax.ShapeDtypeStruct((N, H), _f32),
  )(sp, xs, dinv, b, w)


def _tc3_body(sp_ref, xs_ref, dinv_ref, b_ref, batch_ref, wlin_ref, blin_ref,
              y_ref, gm_ref):
  dinv = dinv_ref[...]
  h = jnp.maximum(dinv * (sp_ref[0, :N] + sp_ref[1, :N] + xs_ref[...]) + b_ref[...],
                  0.0)
  gids = batch_ref[...]
  seg_ids = lax.broadcasted_iota(jnp.int32, (1, G), 1)
  onehot = (gids == seg_ids).astype(_f32)
  seg = lax.dot_general(onehot, h, (((0,), (0,)), ((), ())), **_DOT)
  cnt = lax.dot_general(onehot, jnp.ones((N, 1), _f32),
                        (((0,), (0,)), ((), ())), **_DOT)
  gm = seg / jnp.maximum(cnt, 1.0)
  gm_ref[...] = gm
  y_ref[...] = jnp.dot(gm, wlin_ref[...], **_DOT) + blin_ref[...]


def _tc3_call(sp, xs, dinv, b, batch2d, wlin, blin):
  return pl.pallas_call(
      _tc3_body,
      out_shape=[jax.ShapeDtypeStruct((G, C), _f32),
                 jax.ShapeDtypeStruct((G, H), _f32)],
  )(sp, xs, dinv, b, batch2d, wlin, blin)


def kernel(x, edge_index, batch, W0, b0, W1, b1, W2, b2, Wlin, blin):
  src = edge_index[0]
  dst = edge_index[1]
  ppt = EPT - E // NW
  pad_src = jnp.broadcast_to((jnp.arange(ppt, dtype=jnp.int32) * 89) % N,
                             (NW, ppt))
  pad_dst = jnp.broadcast_to(N + jnp.arange(ppt, dtype=jnp.int32) % (ACC_ROWS - N),
                             (NW, ppt))
  srcg = jnp.concatenate([src.reshape(NW, E // NW), pad_src],
                         axis=1).reshape(NW, CHUNKS, CH)
  dstg = jnp.concatenate([dst.reshape(NW, E // NW), pad_dst],
                         axis=1).reshape(NW, CHUNKS, CH)
  zrows = jnp.zeros((CH, H), _f32)

  degp = _hist_call(dstg).reshape(NW, ACC_ROWS // CH, CH)
  dinv = _tcd_call(degp).reshape(ACC_ROWS, 1)[:N]
  xs = _tc0_call(dinv, x, W0)
  for b, w, relu in ((b0, W1, False), (b1, W2, True)):
    sp = _agg_call(xs, srcg, dstg, zrows)
    xs = _tc_mid_call(relu, sp, xs, dinv, b.reshape(1, H), w)
  sp = _agg_call(xs, srcg, dstg, zrows)
  y, gm = _tc3_call(sp, xs, dinv, b2.reshape(1, H), batch.reshape(N, 1),
                    Wlin, blin.reshape(1, C))
  return (y, gm)
